# Initial kernel scaffold; baseline (speedup 1.0000x reference)
import functools

import jax
import jax.numpy as jnp
from jax import lax
from jax.experimental import pallas as pl
from jax.experimental.pallas import tpu as pltpu
from jax.experimental.pallas import tpu_sc as plsc

N_ROWS = 100000
D = 512
N_SEG = 1024

NC = 2
NS = 16
NW = NC * NS
LANES = 16

CHUNK = 64
ROWS_PER_W = 3128
TRASH = N_SEG


def _sc_partials(x_hbm, idx_hbm, zeros_hbm, out_hbm, xbuf, ibuf, lbuf, acc, sem):
    c = lax.axis_index("c")
    s = lax.axis_index("s")
    w = c * NS + s

    start = w * ROWS_PER_W
    end = jnp.minimum(start + ROWS_PER_W, N_ROWS)
    nrows = end - start
    nchunks = (nrows + CHUNK - 1) // CHUNK

    pltpu.sync_copy(zeros_hbm, acc.at[pl.ds(s * (N_SEG // NS), N_SEG // NS)])
    plsc.subcore_barrier()

    iota = lax.iota(jnp.int32, LANES)

    def body(k, carry):
        k_lo = start + k * CHUNK
        cs = jnp.minimum(k_lo, end - CHUNK)
        pltpu.sync_copy(idx_hbm.at[pl.ds(cs, CHUNK)], ibuf)
        pltpu.sync_copy(x_hbm.at[pl.ds(cs, CHUNK)], xbuf)
        for g in range(CHUNK // LANES):
            v = ibuf[pl.ds(g * LANES, LANES)]
            pos = cs + g * LANES + iota
            lbuf[pl.ds(g * LANES, LANES)] = jnp.where(pos >= k_lo, v, TRASH)
        pltpu.sync_copy(xbuf, acc.at[lbuf], add=True)
        return carry

    lax.fori_loop(0, nchunks, body, 0)

    plsc.subcore_barrier()
    rows = N_SEG // NS
    pltpu.sync_copy(
        acc.at[pl.ds(s * rows, rows)],
        out_hbm.at[pl.ds(c * N_SEG + s * rows, rows)],
    )


def _tc_combine(p0_ref, p1_ref, o_ref):
    o_ref[...] = p0_ref[...] + p1_ref[...]


def kernel(X, graph_idx, n):
    num_segments = n.shape[0]
    idx32 = graph_idx.astype(jnp.int32)
    zeros = jnp.zeros((N_SEG // NS, D), jnp.float32)

    sc = pl.kernel(
        _sc_partials,
        out_type=jax.ShapeDtypeStruct((NC * N_SEG, D), jnp.float32),
        mesh=plsc.VectorSubcoreMesh(core_axis_name="c", subcore_axis_name="s"),
        scratch_types=[
            pltpu.VMEM((CHUNK, D), jnp.float32),
            pltpu.VMEM((CHUNK,), jnp.int32),
            pltpu.VMEM((CHUNK,), jnp.int32),
            pltpu.VMEM_SHARED((N_SEG + 1, D), jnp.float32),
            pltpu.SemaphoreType.DMA,
        ],
    )
    partials = sc(X, idx32, zeros)

    blk = 128
    out = pl.pallas_call(
        _tc_combine,
        grid=(num_segments // blk,),
        in_specs=[
            pl.BlockSpec((blk, D), lambda i: (i, 0)),
            pl.BlockSpec((blk, D), lambda i: (i + num_segments // blk, 0)),
        ],
        out_specs=pl.BlockSpec((blk, D), lambda i: (i, 0)),
        out_shape=jax.ShapeDtypeStruct((num_segments, D), jnp.float32),
    )(partials, partials)
    return out


# speedup vs baseline: 1.1421x; 1.1421x over previous
import jax
import jax.numpy as jnp
from jax import lax
from jax.experimental import pallas as pl
from jax.experimental.pallas import tpu as pltpu
from jax.experimental.pallas import tpu_sc as plsc

N_ROWS = 100000
D = 512
N_SEG = 1024

NC = 2
NS = 16
NW = NC * NS
LANES = 16

SEG_PER_W = N_SEG // NW
CHUNK = 64
NU = D // LANES


def _sc_body(x_hbm, idx_hbm, b_hbm, out_hbm,
             xbufs, ibufs, b_v, acc, xsems, isems):
    c = lax.axis_index("c")
    s = lax.axis_index("s")
    w = c * NS + s
    g0 = w * SEG_PER_W

    pltpu.sync_copy(b_hbm, b_v)
    rs = b_v[pl.ds(g0, LANES)][0]
    re = b_v[pl.ds(g0 + SEG_PER_W, LANES)][0]

    zero = jnp.zeros((LANES,), jnp.float32)

    def zbody(t, carry):
        for u in range(NU):
            acc[t, pl.ds(u * LANES, LANES)] = zero
        return carry

    lax.fori_loop(0, SEG_PER_W, zbody, jnp.int32(0))

    cs0 = pl.multiple_of((rs // 8) * 8, 8)
    nch = (re - cs0 + CHUNK - 1) // CHUNK

    def chunk_start(k):
        return pl.multiple_of(
            jnp.minimum(cs0 + k * CHUNK, N_ROWS - CHUNK), 8)

    def issue(k, p):
        cs = chunk_start(k)
        pltpu.async_copy(x_hbm.at[pl.ds(cs, CHUNK)], xbufs[p], xsems[p])
        pltpu.async_copy(idx_hbm.at[pl.ds(cs, CHUNK)],
                         ibufs[p].at[pl.ds(0, CHUNK)], isems[p])

    for p in range(2):
        @pl.when(p < nch)
        def _():
            issue(jnp.int32(p), p)

    def process(k, p):
        cs = chunk_start(k)
        lo = jnp.maximum(rs, cs0 + k * CHUNK)

        def row(j, carry):
            pos = cs + j

            @pl.when((pos >= lo) & (pos < re))
            def _():
                t = ibufs[p][pl.ds(j, LANES)][0] - g0
                for u in range(NU):
                    col = pl.ds(u * LANES, LANES)
                    acc[t, col] = acc[t, col] + xbufs[p][j, col]
            return carry

        lax.fori_loop(0, CHUNK, row, jnp.int32(0))

    def body(kk, carry):
        for p in range(2):
            k = kk * 2 + p

            @pl.when(k < nch)
            def _():
                pltpu.make_async_copy(
                    x_hbm.at[pl.ds(0, CHUNK)], xbufs[p], xsems[p]).wait()
                pltpu.make_async_copy(
                    idx_hbm.at[pl.ds(0, CHUNK)],
                    ibufs[p].at[pl.ds(0, CHUNK)], isems[p]).wait()
                process(k, p)

                @pl.when(k + 2 < nch)
                def _():
                    issue(k + 2, p)
        return carry

    lax.fori_loop(0, (nch + 1) // 2, body, jnp.int32(0))

    pltpu.sync_copy(acc, out_hbm.at[pl.ds(g0, SEG_PER_W)])


def kernel(X, graph_idx, n):
    num_segments = n.shape[0]
    idx32 = graph_idx.astype(jnp.int32)
    edges = jnp.arange(num_segments + 1, dtype=jnp.int32)
    bounds = jnp.searchsorted(idx32, edges, side="left").astype(jnp.int32)
    bounds = jnp.concatenate(
        [bounds, jnp.full((2 * LANES - 1,), N_ROWS, jnp.int32)])

    sc = pl.kernel(
        _sc_body,
        out_type=jax.ShapeDtypeStruct((N_SEG, D), jnp.float32),
        mesh=plsc.VectorSubcoreMesh(core_axis_name="c", subcore_axis_name="s"),
        scratch_types=[
            [pltpu.VMEM((CHUNK, D), jnp.float32) for _ in range(2)],
            [pltpu.VMEM((CHUNK + LANES,), jnp.int32) for _ in range(2)],
            pltpu.VMEM((N_SEG + 2 * LANES,), jnp.int32),
            pltpu.VMEM((SEG_PER_W, D), jnp.float32),
            [pltpu.SemaphoreType.DMA for _ in range(2)],
            [pltpu.SemaphoreType.DMA for _ in range(2)],
        ],
    )
    return sc(X, idx32, bounds)


# speedup vs baseline: 3.5237x; 3.0855x over previous
import jax
import jax.numpy as jnp
from jax import lax
from jax.experimental import pallas as pl
from jax.experimental.pallas import tpu as pltpu
from jax.experimental.pallas import tpu_sc as plsc

N_ROWS = 100000
D = 512
N_SEG = 1024

NC = 2
NS = 16
NW = NC * NS
LANES = 16

SEG_PER_W = N_SEG // NW
CHUNK = 64
NU = D // LANES


def _sc_body(x_hbm, idx_hbm, b_hbm, out_hbm,
             xbufs, ibufs, b_v, acc, xsems, isems):
    c = lax.axis_index("c")
    s = lax.axis_index("s")
    w = c * NS + s
    g0 = w * SEG_PER_W

    pltpu.sync_copy(b_hbm, b_v)
    rs = b_v[pl.ds(g0, LANES)][0]
    re = b_v[pl.ds(g0 + SEG_PER_W, LANES)][0]

    zero = jnp.zeros((LANES,), jnp.float32)

    def zbody(t, carry):
        for u in range(NU):
            acc[t, pl.ds(u * LANES, LANES)] = zero
        return carry

    lax.fori_loop(0, SEG_PER_W, zbody, jnp.int32(0))

    cs0 = pl.multiple_of((rs // 8) * 8, 8)
    nch = (re - cs0 + CHUNK - 1) // CHUNK

    def chunk_start(k):
        return pl.multiple_of(
            jnp.minimum(cs0 + k * CHUNK, N_ROWS - CHUNK), 8)

    def issue(k, p):
        cs = chunk_start(k)
        pltpu.async_copy(x_hbm.at[pl.ds(cs, CHUNK)], xbufs[p], xsems[p])
        pltpu.async_copy(idx_hbm.at[pl.ds(cs, CHUNK)],
                         ibufs[p].at[pl.ds(0, CHUNK)], isems[p])

    for p in range(2):
        @pl.when(p < nch)
        def _():
            issue(jnp.int32(p), p)

    def process(k, p):
        cs = chunk_start(k)
        jlo = jnp.maximum(rs, cs0 + k * CHUNK) - cs
        jhi = jnp.minimum(re, cs + CHUNK) - cs

        tlo = ibufs[p][pl.ds(jlo, LANES)][0]
        thi = ibufs[p][pl.ds(jnp.maximum(jhi - 1, jlo), LANES)][0]
        npieces = jnp.where(jhi > jlo, thi - tlo + 1, 0)

        def piece(i, j):
            done = j >= jhi
            jj = jnp.minimum(j, CHUNK - 1)
            t_raw = ibufs[p][pl.ds(jj, LANES)][0] - g0
            t = jnp.clip(t_raw, 0, SEG_PER_W - 1)
            e = b_v[pl.ds(g0 + t + 1, LANES)][0] - cs
            jend = jnp.where(done, j, jnp.minimum(e, jhi))
            av = tuple(acc[t, pl.ds(u * LANES, LANES)] for u in range(NU))

            def row(r, av):
                return tuple(
                    av[u] + xbufs[p][r, pl.ds(u * LANES, LANES)]
                    for u in range(NU))

            av = lax.fori_loop(j, jend, row, av)
            for u in range(NU):
                acc[t, pl.ds(u * LANES, LANES)] = av[u]
            return jend

        lax.fori_loop(0, npieces, piece, jlo)

    def body(kk, carry):
        for p in range(2):
            k = kk * 2 + p

            @pl.when(k < nch)
            def _():
                pltpu.make_async_copy(
                    x_hbm.at[pl.ds(0, CHUNK)], xbufs[p], xsems[p]).wait()
                pltpu.make_async_copy(
                    idx_hbm.at[pl.ds(0, CHUNK)],
                    ibufs[p].at[pl.ds(0, CHUNK)], isems[p]).wait()
                process(k, p)

                @pl.when(k + 2 < nch)
                def _():
                    issue(k + 2, p)
        return carry

    lax.fori_loop(0, (nch + 1) // 2, body, jnp.int32(0))

    pltpu.sync_copy(acc, out_hbm.at[pl.ds(g0, SEG_PER_W)])


def kernel(X, graph_idx, n):
    num_segments = n.shape[0]
    idx32 = graph_idx.astype(jnp.int32)
    edges = jnp.arange(num_segments + 1, dtype=jnp.int32)
    bounds = jnp.searchsorted(idx32, edges, side="left").astype(jnp.int32)
    bounds = jnp.concatenate(
        [bounds, jnp.full((2 * LANES - 1,), N_ROWS, jnp.int32)])

    sc = pl.kernel(
        _sc_body,
        out_type=jax.ShapeDtypeStruct((N_SEG, D), jnp.float32),
        mesh=plsc.VectorSubcoreMesh(core_axis_name="c", subcore_axis_name="s"),
        scratch_types=[
            [pltpu.VMEM((CHUNK, D), jnp.float32) for _ in range(2)],
            [pltpu.VMEM((CHUNK + LANES,), jnp.int32) for _ in range(2)],
            pltpu.VMEM((N_SEG + 2 * LANES,), jnp.int32),
            pltpu.VMEM((SEG_PER_W, D), jnp.float32),
            [pltpu.SemaphoreType.DMA for _ in range(2)],
            [pltpu.SemaphoreType.DMA for _ in range(2)],
        ],
    )
    return sc(X, idx32, bounds)


# speedup vs baseline: 3.7965x; 1.0774x over previous
import jax
import jax.numpy as jnp
from jax import lax
from jax.experimental import pallas as pl
from jax.experimental.pallas import tpu as pltpu
from jax.experimental.pallas import tpu_sc as plsc

N_ROWS = 100000
D = 512
N_SEG = 1024

NC = 2
NS = 16
NW = NC * NS
LANES = 16

SEG_PER_W = N_SEG // NW
CHUNK = 64
NSLOT = 3
NU = D // LANES


def _sc_body(x_hbm, idx_hbm, b_hbm, out_hbm,
             xbufs, ibufs, b_v, acc, xsems, isems):
    c = lax.axis_index("c")
    s = lax.axis_index("s")
    w = c * NS + s
    g0 = w * SEG_PER_W

    pltpu.sync_copy(b_hbm, b_v)
    rs = b_v[pl.ds(g0, LANES)][0]
    re = b_v[pl.ds(g0 + SEG_PER_W, LANES)][0]

    zero = jnp.zeros((LANES,), jnp.float32)

    def zbody(t, carry):
        for u in range(NU):
            acc[t, pl.ds(u * LANES, LANES)] = zero
        return carry

    lax.fori_loop(0, SEG_PER_W, zbody, jnp.int32(0))

    cs0 = pl.multiple_of((rs // 8) * 8, 8)
    nch = (re - cs0 + CHUNK - 1) // CHUNK

    def chunk_start(k):
        return pl.multiple_of(
            jnp.minimum(cs0 + k * CHUNK, N_ROWS - CHUNK), 8)

    def issue(k, p):
        cs = chunk_start(k)
        pltpu.async_copy(x_hbm.at[pl.ds(cs, CHUNK)], xbufs[p], xsems[p])
        pltpu.async_copy(idx_hbm.at[pl.ds(cs, CHUNK)],
                         ibufs[p].at[pl.ds(0, CHUNK)], isems[p])

    for p in range(NSLOT):
        @pl.when(p < nch)
        def _():
            issue(jnp.int32(p), p)

    def process(k, p):
        cs = chunk_start(k)
        jlo = jnp.maximum(rs, cs0 + k * CHUNK) - cs
        jhi = jnp.minimum(re, cs + CHUNK) - cs

        tlo = ibufs[p][pl.ds(jlo, LANES)][0]
        thi = ibufs[p][pl.ds(jnp.maximum(jhi - 1, jlo), LANES)][0]
        npieces = jnp.where(jhi > jlo, thi - tlo + 1, 0)

        def piece(i, j):
            done = j >= jhi
            jj = jnp.minimum(j, CHUNK - 1)
            t_raw = ibufs[p][pl.ds(jj, LANES)][0] - g0
            t = jnp.clip(t_raw, 0, SEG_PER_W - 1)
            e = b_v[pl.ds(g0 + t + 1, LANES)][0] - cs
            jend = jnp.where(done, j, jnp.minimum(e, jhi))
            av = tuple(acc[t, pl.ds(u * LANES, LANES)] for u in range(NU))

            def row(r, av):
                return tuple(
                    av[u] + xbufs[p][r, pl.ds(u * LANES, LANES)]
                    for u in range(NU))

            av = lax.fori_loop(j, jend, row, av)
            for u in range(NU):
                acc[t, pl.ds(u * LANES, LANES)] = av[u]
            return jend

        lax.fori_loop(0, npieces, piece, jlo)

    def body(kk, carry):
        for p in range(NSLOT):
            k = kk * NSLOT + p

            @pl.when(k < nch)
            def _():
                pltpu.make_async_copy(
                    x_hbm.at[pl.ds(0, CHUNK)], xbufs[p], xsems[p]).wait()
                pltpu.make_async_copy(
                    idx_hbm.at[pl.ds(0, CHUNK)],
                    ibufs[p].at[pl.ds(0, CHUNK)], isems[p]).wait()

                @pl.when(k + NSLOT < nch)
                def _():
                    issue(k + NSLOT, p)

                process(k, p)
        return carry

    lax.fori_loop(0, (nch + NSLOT - 1) // NSLOT, body, jnp.int32(0))

    pltpu.sync_copy(acc, out_hbm.at[pl.ds(g0, SEG_PER_W)])


def kernel(X, graph_idx, n):
    num_segments = n.shape[0]
    idx32 = graph_idx.astype(jnp.int32)
    edges = jnp.arange(num_segments + 1, dtype=jnp.int32)
    bounds = jnp.searchsorted(idx32, edges, side="left").astype(jnp.int32)
    bounds = jnp.concatenate(
        [bounds, jnp.full((2 * LANES - 1,), N_ROWS, jnp.int32)])

    sc = pl.kernel(
        _sc_body,
        out_type=jax.ShapeDtypeStruct((N_SEG, D), jnp.float32),
        mesh=plsc.VectorSubcoreMesh(core_axis_name="c", subcore_axis_name="s"),
        scratch_types=[
            [pltpu.VMEM((CHUNK, D), jnp.float32) for _ in range(NSLOT)],
            [pltpu.VMEM((CHUNK + LANES,), jnp.int32) for _ in range(NSLOT)],
            pltpu.VMEM((N_SEG + 2 * LANES,), jnp.int32),
            pltpu.VMEM((SEG_PER_W, D), jnp.float32),
            [pltpu.SemaphoreType.DMA for _ in range(NSLOT)],
            [pltpu.SemaphoreType.DMA for _ in range(NSLOT)],
        ],
    )
    return sc(X, idx32, bounds)
